# baseline (speedup 1.0000x reference)
import functools

import jax
import jax.numpy as jnp
from jax.experimental import pallas as pl
from jax.experimental.pallas import tpu as pltpu

_NCLS = 10
_TRUNC = None


def _shift3(x):
    z = jnp.zeros((1, x.shape[1]), x.dtype)
    xm = jnp.concatenate([z, x[:-1]], axis=0)
    xp = jnp.concatenate([x[1:], z], axis=0)
    return xm, x, xp


def _conv_row(x_ref, w_ref, b_ref, r, pre):
    if pre:
        xc = x_ref[pl.ds(r - 1, 1)][0]
    else:
        xw = x_ref[pl.ds(r - 1, 3)]
        parts = []
        for dy in range(3):
            parts.extend(_shift3(xw[dy]))
        xc = jnp.concatenate(parts, axis=1)
    acc = jnp.dot(xc, w_ref[...], preferred_element_type=jnp.float32)
    return jnp.maximum(acc + b_ref[...], 0.0)


def _conv_plain_kernel(x_ref, w_ref, b_ref, o_ref, *, hh, wp, pre):
    tm = x_ref.shape[1]
    cout = o_ref.shape[2]
    col = jax.lax.broadcasted_iota(jnp.int32, (tm, 1), 0) % wp
    valid = jnp.logical_and(col > 0, col < wp - 1)
    zrow = jnp.zeros((1, tm, cout), o_ref.dtype)
    o_ref[pl.ds(0, 1)] = zrow
    o_ref[pl.ds(hh + 1, 1)] = zrow

    def body(r, _):
        acc = _conv_row(x_ref, w_ref, b_ref, r, pre)
        out = jnp.where(valid, acc, 0.0).astype(o_ref.dtype)
        o_ref[pl.ds(r, 1)] = out[None]
        return _

    jax.lax.fori_loop(1, hh + 1, body, None)


def _conv_pool_kernel(x_ref, w_ref, b_ref, o_ref, *, hh, wp, btile):
    w_valid = wp - 2
    wo = w_valid // 2
    wpn = wo + 2
    cout = o_ref.shape[2]
    tmo = o_ref.shape[1]
    ho = hh // 2
    zrow = jnp.zeros((1, tmo, cout), o_ref.dtype)
    o_ref[pl.ds(0, 1)] = zrow
    o_ref[pl.ds(ho + 1, 1)] = zrow

    def body(i, _):
        c0 = _conv_row(x_ref, w_ref, b_ref, 2 * i - 1, False)
        c1 = _conv_row(x_ref, w_ref, b_ref, 2 * i, False)
        v = jnp.maximum(c0, c1).reshape(btile, wp, cout)
        v = v[:, 1:1 + w_valid].reshape(btile, wo, 2, cout).max(axis=2)
        zc = jnp.zeros((btile, 1, cout), v.dtype)
        v = jnp.concatenate([zc, v, zc], axis=1).reshape(tmo, cout)
        o_ref[pl.ds(i, 1)] = v[None].astype(o_ref.dtype)
        return _

    jax.lax.fori_loop(1, ho + 1, body, None)


def _conv_fc_kernel(x_ref, w_ref, b_ref, w1_ref, b1_ref, w2_ref, b2_ref,
                    o_ref, *, wp, btile):
    cout = w_ref.shape[1]
    c0 = _conv_row(x_ref, w_ref, b_ref, 1, False)
    c1 = _conv_row(x_ref, w_ref, b_ref, 2, False)
    v = jnp.maximum(c0, c1).reshape(btile, wp, cout)
    feat = v[:, 1:3].max(axis=1).astype(jnp.bfloat16)
    h = jnp.dot(feat, w1_ref[...], preferred_element_type=jnp.float32)
    h = jnp.maximum(h + b1_ref[...], 0.0).astype(jnp.bfloat16)
    logits = jnp.dot(h, w2_ref[...], preferred_element_type=jnp.float32)
    o_ref[...] = logits + b2_ref[...]


_CFG = [
    (32, 128, False, 16, True),
    (32, 128, True, 16, False),
    (16, 128, False, 16, False),
    (16, 128, True, 16, False),
    (8, 256, False, 32, False),
    (8, 256, False, 32, False),
    (8, 256, True, 32, False),
    (4, 512, False, 32, False),
    (4, 512, False, 32, False),
    (4, 512, True, 32, False),
    (2, 512, False, 32, False),
    (2, 512, False, 32, False),
    (2, 512, True, 32, False),
]

_VMEM = dict(vmem_limit_bytes=64 * 1024 * 1024)


def _conv_call(x, wcat, bias, *, hh, cout, pool, btile, pre=False):
    hp, m, _ = x.shape
    wp = hh + 2
    nb = m // wp
    btile = min(btile, nb)
    n_bt = nb // btile
    tm = btile * wp
    if pool:
        ho = hh // 2
        wo = (wp - 2) // 2
        wpn = wo + 2
        out_shape = jax.ShapeDtypeStruct((ho + 2, nb * wpn, cout), jnp.bfloat16)
        out_spec = pl.BlockSpec((ho + 2, btile * wpn, cout), lambda i: (0, i, 0))
        kern = functools.partial(_conv_pool_kernel, hh=hh, wp=wp, btile=btile)
    else:
        out_shape = jax.ShapeDtypeStruct((hh + 2, m, cout), jnp.bfloat16)
        out_spec = pl.BlockSpec((hh + 2, tm, cout), lambda i: (0, i, 0))
        kern = functools.partial(_conv_plain_kernel, hh=hh, wp=wp, pre=pre)
    return pl.pallas_call(
        kern,
        out_shape=out_shape,
        grid=(n_bt,),
        in_specs=[
            pl.BlockSpec((hp, tm, x.shape[2]), lambda i: (0, i, 0)),
            pl.BlockSpec(wcat.shape, lambda i: (0, 0)),
            pl.BlockSpec(bias.shape, lambda i: (0, 0)),
        ],
        out_specs=out_spec,
        compiler_params=pltpu.CompilerParams(
            dimension_semantics=("parallel",), **_VMEM),
    )(x, wcat, bias)


def _conv_fc_call(x, wcat, bias, w1, b1, w2, b2, *, btile):
    hp, m, cin = x.shape
    wp = 4
    nb = m // wp
    btile = min(btile, nb)
    n_bt = nb // btile
    tm = btile * wp
    ncp = w2.shape[1]
    kern = functools.partial(_conv_fc_kernel, wp=wp, btile=btile)
    return pl.pallas_call(
        kern,
        out_shape=jax.ShapeDtypeStruct((nb, ncp), jnp.float32),
        grid=(n_bt,),
        in_specs=[
            pl.BlockSpec((hp, tm, cin), lambda i: (0, i, 0)),
            pl.BlockSpec(wcat.shape, lambda i: (0, 0)),
            pl.BlockSpec(bias.shape, lambda i: (0, 0)),
            pl.BlockSpec(w1.shape, lambda i: (0, 0)),
            pl.BlockSpec(b1.shape, lambda i: (0, 0)),
            pl.BlockSpec(w2.shape, lambda i: (0, 0)),
            pl.BlockSpec(b2.shape, lambda i: (0, 0)),
        ],
        out_specs=pl.BlockSpec((btile, ncp), lambda i: (i, 0)),
        compiler_params=pltpu.CompilerParams(
            dimension_semantics=("parallel",), **_VMEM),
    )(x, wcat, bias, w1, b1, w2, b2)


def _prep_x(x_nchw):
    b = x_nchw.shape[0]
    x = jnp.transpose(x_nchw, (2, 0, 3, 1)).astype(jnp.bfloat16)
    x = jnp.pad(x, ((1, 1), (0, 0), (2, 2), (0, 5)))
    taps = [x[dy:dy + 32, :, dx:dx + 34, :]
            for dy in range(3) for dx in range(3)]
    xc = jnp.concatenate(taps, axis=-1)
    xc = jnp.pad(xc, ((0, 0), (0, 0), (0, 0), (0, 56)))
    return xc.reshape(32, b * 34, 128)


def _prep_w(w, pre):
    if pre:
        w = jnp.pad(w, ((0, 0), (0, 5), (0, 0)))
        w = w.reshape(72, w.shape[2])
        return jnp.pad(w, ((0, 56), (0, 0)))
    return w.reshape(9 * w.shape[1], w.shape[2])


def kernel(x_nchw, conv0_w, conv0_b, conv1_w, conv1_b, conv2_w, conv2_b,
           conv3_w, conv3_b, conv4_w, conv4_b, conv5_w, conv5_b,
           conv6_w, conv6_b, conv7_w, conv7_b, conv8_w, conv8_b,
           conv9_w, conv9_b, conv10_w, conv10_b, conv11_w, conv11_b,
           conv12_w, conv12_b, fc_w1, fc_b1, fc_w2, fc_b2):
    ws = [conv0_w, conv1_w, conv2_w, conv3_w, conv4_w, conv5_w, conv6_w,
          conv7_w, conv8_w, conv9_w, conv10_w, conv11_w, conv12_w]
    bs = [conv0_b, conv1_b, conv2_b, conv3_b, conv4_b, conv5_b, conv6_b,
          conv7_b, conv8_b, conv9_b, conv10_b, conv11_b, conv12_b]
    x = _prep_x(x_nchw)
    for i, (hh, cout, pool, btile, pre) in enumerate(_CFG):
        if i == _TRUNC:
            return x[1, :128, :10]
        wcat = _prep_w(ws[i], pre)
        if i == len(_CFG) - 1:
            logits = _conv_fc_call(x, wcat, bs[i], fc_w1, fc_b1, fc_w2, fc_b2,
                                   btile=btile)
            return logits[:, :_NCLS]
        x = _conv_call(x, wcat, bs[i], hh=hh, cout=cout,
                       pool=pool, btile=btile, pre=pre)


# speedup vs baseline: 1.7323x; 1.7323x over previous
import functools

import jax
import jax.numpy as jnp
from jax.experimental import pallas as pl
from jax.experimental.pallas import tpu as pltpu

_NCLS = 10
_TRUNC = None


def _shift3(x):
    z = jnp.zeros((1, x.shape[1]), x.dtype)
    xm = jnp.concatenate([z, x[:-1]], axis=0)
    xp = jnp.concatenate([x[1:], z], axis=0)
    return xm, x, xp


def _conv_row(x_ref, w_ref, b_ref, r, pre):
    if pre:
        xc = x_ref[pl.ds(r - 1, 1)][0]
    else:
        xw = x_ref[pl.ds(r - 1, 3)]
        parts = []
        for dy in range(3):
            parts.extend(_shift3(xw[dy]))
        xc = jnp.concatenate(parts, axis=1)
    acc = jnp.dot(xc, w_ref[...], preferred_element_type=jnp.float32)
    return jnp.maximum(acc + b_ref[...], 0.0)


def _conv_plain_kernel(x_ref, w_ref, b_ref, o_ref, *, hh, wp, pre):
    tm = x_ref.shape[1]
    cout = o_ref.shape[2]
    col = jax.lax.broadcasted_iota(jnp.int32, (tm, 1), 0) % wp
    valid = jnp.logical_and(col > 0, col < wp - 1)
    zrow = jnp.zeros((1, tm, cout), o_ref.dtype)
    o_ref[pl.ds(0, 1)] = zrow
    o_ref[pl.ds(hh + 1, 1)] = zrow

    def body(r, _):
        acc = _conv_row(x_ref, w_ref, b_ref, r, pre)
        out = jnp.where(valid, acc, 0.0).astype(o_ref.dtype)
        o_ref[pl.ds(r, 1)] = out[None]
        return _

    jax.lax.fori_loop(1, hh + 1, body, None)


def _conv_pool_kernel(x_ref, w_ref, b_ref, o_ref, *, hh, wp, btile):
    w_valid = wp - 2
    wo = w_valid // 2
    wpn = wo + 2
    cout = o_ref.shape[2]
    tmo = o_ref.shape[1]
    ho = hh // 2
    zrow = jnp.zeros((1, tmo, cout), o_ref.dtype)
    o_ref[pl.ds(0, 1)] = zrow
    o_ref[pl.ds(ho + 1, 1)] = zrow

    def body(i, _):
        c0 = _conv_row(x_ref, w_ref, b_ref, 2 * i - 1, False)
        c1 = _conv_row(x_ref, w_ref, b_ref, 2 * i, False)
        v = jnp.maximum(c0, c1).reshape(btile, wp, cout)
        v = v[:, 1:1 + w_valid].reshape(btile, wo, 2, cout).max(axis=2)
        zc = jnp.zeros((btile, 1, cout), v.dtype)
        v = jnp.concatenate([zc, v, zc], axis=1).reshape(tmo, cout)
        o_ref[pl.ds(i, 1)] = v[None].astype(o_ref.dtype)
        return _

    jax.lax.fori_loop(1, ho + 1, body, None)


def _conv_fc_kernel(x_ref, w_ref, b_ref, w1_ref, b1_ref, w2_ref, b2_ref,
                    o_ref, *, wp, btile):
    cout = w_ref.shape[1]
    c0 = _conv_row(x_ref, w_ref, b_ref, 1, False)
    c1 = _conv_row(x_ref, w_ref, b_ref, 2, False)
    v = jnp.maximum(c0, c1).reshape(btile, wp, cout)
    feat = v[:, 1:3].max(axis=1).astype(jnp.bfloat16)
    h = jnp.dot(feat, w1_ref[...], preferred_element_type=jnp.float32)
    h = jnp.maximum(h + b1_ref[...], 0.0).astype(jnp.bfloat16)
    logits = jnp.dot(h, w2_ref[...], preferred_element_type=jnp.float32)
    o_ref[...] = logits + b2_ref[...]


_CFG = [
    (32, 128, False, 16, False),
    (32, 128, True, 16, False),
    (16, 128, False, 16, False),
    (16, 128, True, 16, False),
    (8, 256, False, 32, False),
    (8, 256, False, 32, False),
    (8, 256, True, 32, False),
    (4, 512, False, 32, False),
    (4, 512, False, 32, False),
    (4, 512, True, 32, False),
    (2, 512, False, 32, False),
    (2, 512, False, 32, False),
    (2, 512, True, 32, False),
]

_VMEM = dict(vmem_limit_bytes=64 * 1024 * 1024)


def _conv_call(x, wcat, bias, *, hh, cout, pool, btile, pre=False):
    hp, m, _ = x.shape
    wp = hh + 2
    nb = m // wp
    btile = min(btile, nb)
    n_bt = nb // btile
    tm = btile * wp
    if pool:
        ho = hh // 2
        wo = (wp - 2) // 2
        wpn = wo + 2
        out_shape = jax.ShapeDtypeStruct((ho + 2, nb * wpn, cout), jnp.bfloat16)
        out_spec = pl.BlockSpec((ho + 2, btile * wpn, cout), lambda i: (0, i, 0))
        kern = functools.partial(_conv_pool_kernel, hh=hh, wp=wp, btile=btile)
    else:
        out_shape = jax.ShapeDtypeStruct((hh + 2, m, cout), jnp.bfloat16)
        out_spec = pl.BlockSpec((hh + 2, tm, cout), lambda i: (0, i, 0))
        kern = functools.partial(_conv_plain_kernel, hh=hh, wp=wp, pre=pre)
    return pl.pallas_call(
        kern,
        out_shape=out_shape,
        grid=(n_bt,),
        in_specs=[
            pl.BlockSpec((hp, tm, x.shape[2]), lambda i: (0, i, 0)),
            pl.BlockSpec(wcat.shape, lambda i: (0, 0)),
            pl.BlockSpec(bias.shape, lambda i: (0, 0)),
        ],
        out_specs=out_spec,
        compiler_params=pltpu.CompilerParams(
            dimension_semantics=("parallel",), **_VMEM),
    )(x, wcat, bias)


def _conv_fc_call(x, wcat, bias, w1, b1, w2, b2, *, btile):
    hp, m, cin = x.shape
    wp = 4
    nb = m // wp
    btile = min(btile, nb)
    n_bt = nb // btile
    tm = btile * wp
    ncp = w2.shape[1]
    kern = functools.partial(_conv_fc_kernel, wp=wp, btile=btile)
    return pl.pallas_call(
        kern,
        out_shape=jax.ShapeDtypeStruct((nb, ncp), jnp.float32),
        grid=(n_bt,),
        in_specs=[
            pl.BlockSpec((hp, tm, cin), lambda i: (0, i, 0)),
            pl.BlockSpec(wcat.shape, lambda i: (0, 0)),
            pl.BlockSpec(bias.shape, lambda i: (0, 0)),
            pl.BlockSpec(w1.shape, lambda i: (0, 0)),
            pl.BlockSpec(b1.shape, lambda i: (0, 0)),
            pl.BlockSpec(w2.shape, lambda i: (0, 0)),
            pl.BlockSpec(b2.shape, lambda i: (0, 0)),
        ],
        out_specs=pl.BlockSpec((btile, ncp), lambda i: (i, 0)),
        compiler_params=pltpu.CompilerParams(
            dimension_semantics=("parallel",), **_VMEM),
    )(x, wcat, bias, w1, b1, w2, b2)


def _prep_x(x_nchw):
    b = x_nchw.shape[0]
    x = jnp.transpose(x_nchw, (0, 2, 3, 1)).astype(jnp.bfloat16)
    x = jnp.pad(x, ((0, 0), (0, 0), (1, 1), (0, 5)))
    x = jnp.transpose(x, (1, 0, 2, 3)).reshape(32, b * 34, 8)
    return jnp.pad(x, ((1, 1), (0, 0), (0, 0)))


def _prep_w(w, pre):
    del pre
    if w.shape[1] == 3:
        w = jnp.pad(w, ((0, 0), (0, 5), (0, 0)))
    return w.reshape(9 * w.shape[1], w.shape[2])


def kernel(x_nchw, conv0_w, conv0_b, conv1_w, conv1_b, conv2_w, conv2_b,
           conv3_w, conv3_b, conv4_w, conv4_b, conv5_w, conv5_b,
           conv6_w, conv6_b, conv7_w, conv7_b, conv8_w, conv8_b,
           conv9_w, conv9_b, conv10_w, conv10_b, conv11_w, conv11_b,
           conv12_w, conv12_b, fc_w1, fc_b1, fc_w2, fc_b2):
    ws = [conv0_w, conv1_w, conv2_w, conv3_w, conv4_w, conv5_w, conv6_w,
          conv7_w, conv8_w, conv9_w, conv10_w, conv11_w, conv12_w]
    bs = [conv0_b, conv1_b, conv2_b, conv3_b, conv4_b, conv5_b, conv6_b,
          conv7_b, conv8_b, conv9_b, conv10_b, conv11_b, conv12_b]
    x = _prep_x(x_nchw)
    for i, (hh, cout, pool, btile, pre) in enumerate(_CFG):
        if i == _TRUNC:
            return x[1, :128, :10]
        wcat = _prep_w(ws[i], pre)
        if i == len(_CFG) - 1:
            logits = _conv_fc_call(x, wcat, bs[i], fc_w1, fc_b1, fc_w2, fc_b2,
                                   btile=btile)
            return logits[:, :_NCLS]
        x = _conv_call(x, wcat, bs[i], hh=hh, cout=cout,
                       pool=pool, btile=btile, pre=pre)


# speedup vs baseline: 2.0718x; 1.1960x over previous
import functools

import jax
import jax.numpy as jnp
from jax.experimental import pallas as pl
from jax.experimental.pallas import tpu as pltpu

_NCLS = 10
_TRUNC = None


def _shift3(x):
    z = jnp.zeros((1, x.shape[1]), x.dtype)
    xm = jnp.concatenate([z, x[:-1]], axis=0)
    xp = jnp.concatenate([x[1:], z], axis=0)
    return xm, x, xp


def _conv_row(x_ref, w_ref, b_ref, r, pre):
    if pre:
        xc = x_ref[pl.ds(r - 1, 1)][0]
    else:
        xw = x_ref[pl.ds(r - 1, 3)]
        parts = []
        for dy in range(3):
            parts.extend(_shift3(xw[dy]))
        xc = jnp.concatenate(parts, axis=1)
    acc = jnp.dot(xc, w_ref[...], preferred_element_type=jnp.float32)
    return jnp.maximum(acc + b_ref[...], 0.0)


def _conv_plain_kernel(x_ref, w_ref, b_ref, o_ref, *, hh, wp, pre):
    tm = x_ref.shape[1]
    cout = o_ref.shape[2]
    col = jax.lax.broadcasted_iota(jnp.int32, (tm, 1), 0) % wp
    valid = jnp.logical_and(col > 0, col < wp - 1)
    zrow = jnp.zeros((1, tm, cout), o_ref.dtype)
    o_ref[pl.ds(0, 1)] = zrow
    o_ref[pl.ds(hh + 1, 1)] = zrow

    for r in range(1, hh + 1):
        acc = _conv_row(x_ref, w_ref, b_ref, r, pre)
        out = jnp.where(valid, acc, 0.0).astype(o_ref.dtype)
        o_ref[pl.ds(r, 1)] = out[None]


def _conv_pool_kernel(x_ref, w_ref, b_ref, o_ref, *, hh, wp, btile):
    w_valid = wp - 2
    wo = w_valid // 2
    wpn = wo + 2
    cout = o_ref.shape[2]
    tmo = o_ref.shape[1]
    ho = hh // 2
    zrow = jnp.zeros((1, tmo, cout), o_ref.dtype)
    o_ref[pl.ds(0, 1)] = zrow
    o_ref[pl.ds(ho + 1, 1)] = zrow

    for i in range(1, ho + 1):
        c0 = _conv_row(x_ref, w_ref, b_ref, 2 * i - 1, False)
        c1 = _conv_row(x_ref, w_ref, b_ref, 2 * i, False)
        v = jnp.maximum(c0, c1).reshape(btile, wp, cout)
        v = v[:, 1:1 + w_valid].reshape(btile, wo, 2, cout).max(axis=2)
        zc = jnp.zeros((btile, 1, cout), v.dtype)
        v = jnp.concatenate([zc, v, zc], axis=1).reshape(tmo, cout)
        o_ref[pl.ds(i, 1)] = v[None].astype(o_ref.dtype)


def _conv_fc_kernel(x_ref, w_ref, b_ref, w1_ref, b1_ref, w2_ref, b2_ref,
                    o_ref, *, wp, btile):
    cout = w_ref.shape[1]
    c0 = _conv_row(x_ref, w_ref, b_ref, 1, False)
    c1 = _conv_row(x_ref, w_ref, b_ref, 2, False)
    v = jnp.maximum(c0, c1).reshape(btile, wp, cout)
    feat = v[:, 1:3].max(axis=1).astype(jnp.bfloat16)
    h = jnp.dot(feat, w1_ref[...], preferred_element_type=jnp.float32)
    h = jnp.maximum(h + b1_ref[...], 0.0).astype(jnp.bfloat16)
    logits = jnp.dot(h, w2_ref[...], preferred_element_type=jnp.float32)
    o_ref[...] = logits + b2_ref[...]


_CFG = [
    (32, 128, False, 16, False),
    (32, 128, True, 16, False),
    (16, 128, False, 16, False),
    (16, 128, True, 16, False),
    (8, 256, False, 32, False),
    (8, 256, False, 32, False),
    (8, 256, True, 32, False),
    (4, 512, False, 32, False),
    (4, 512, False, 32, False),
    (4, 512, True, 32, False),
    (2, 512, False, 32, False),
    (2, 512, False, 32, False),
    (2, 512, True, 32, False),
]

_VMEM = dict(vmem_limit_bytes=64 * 1024 * 1024)


def _conv_call(x, wcat, bias, *, hh, cout, pool, btile, pre=False):
    hp, m, _ = x.shape
    wp = hh + 2
    nb = m // wp
    btile = min(btile, nb)
    n_bt = nb // btile
    tm = btile * wp
    if pool:
        ho = hh // 2
        wo = (wp - 2) // 2
        wpn = wo + 2
        out_shape = jax.ShapeDtypeStruct((ho + 2, nb * wpn, cout), jnp.bfloat16)
        out_spec = pl.BlockSpec((ho + 2, btile * wpn, cout), lambda i: (0, i, 0))
        kern = functools.partial(_conv_pool_kernel, hh=hh, wp=wp, btile=btile)
    else:
        out_shape = jax.ShapeDtypeStruct((hh + 2, m, cout), jnp.bfloat16)
        out_spec = pl.BlockSpec((hh + 2, tm, cout), lambda i: (0, i, 0))
        kern = functools.partial(_conv_plain_kernel, hh=hh, wp=wp, pre=pre)
    return pl.pallas_call(
        kern,
        out_shape=out_shape,
        grid=(n_bt,),
        in_specs=[
            pl.BlockSpec((hp, tm, x.shape[2]), lambda i: (0, i, 0)),
            pl.BlockSpec(wcat.shape, lambda i: (0, 0)),
            pl.BlockSpec(bias.shape, lambda i: (0, 0)),
        ],
        out_specs=out_spec,
        compiler_params=pltpu.CompilerParams(
            dimension_semantics=("parallel",), **_VMEM),
    )(x, wcat, bias)


def _conv_fc_call(x, wcat, bias, w1, b1, w2, b2, *, btile):
    hp, m, cin = x.shape
    wp = 4
    nb = m // wp
    btile = min(btile, nb)
    n_bt = nb // btile
    tm = btile * wp
    ncp = w2.shape[1]
    kern = functools.partial(_conv_fc_kernel, wp=wp, btile=btile)
    return pl.pallas_call(
        kern,
        out_shape=jax.ShapeDtypeStruct((nb, ncp), jnp.float32),
        grid=(n_bt,),
        in_specs=[
            pl.BlockSpec((hp, tm, cin), lambda i: (0, i, 0)),
            pl.BlockSpec(wcat.shape, lambda i: (0, 0)),
            pl.BlockSpec(bias.shape, lambda i: (0, 0)),
            pl.BlockSpec(w1.shape, lambda i: (0, 0)),
            pl.BlockSpec(b1.shape, lambda i: (0, 0)),
            pl.BlockSpec(w2.shape, lambda i: (0, 0)),
            pl.BlockSpec(b2.shape, lambda i: (0, 0)),
        ],
        out_specs=pl.BlockSpec((btile, ncp), lambda i: (i, 0)),
        compiler_params=pltpu.CompilerParams(
            dimension_semantics=("parallel",), **_VMEM),
    )(x, wcat, bias, w1, b1, w2, b2)


def _prep_x(x_nchw):
    b = x_nchw.shape[0]
    x = jnp.transpose(x_nchw, (0, 2, 3, 1)).astype(jnp.bfloat16)
    x = jnp.pad(x, ((0, 0), (0, 0), (1, 1), (0, 5)))
    x = jnp.transpose(x, (1, 0, 2, 3)).reshape(32, b * 34, 8)
    return jnp.pad(x, ((1, 1), (0, 0), (0, 0)))


def _prep_w(w, pre):
    del pre
    if w.shape[1] == 3:
        w = jnp.pad(w, ((0, 0), (0, 5), (0, 0)))
    return w.reshape(9 * w.shape[1], w.shape[2])


def kernel(x_nchw, conv0_w, conv0_b, conv1_w, conv1_b, conv2_w, conv2_b,
           conv3_w, conv3_b, conv4_w, conv4_b, conv5_w, conv5_b,
           conv6_w, conv6_b, conv7_w, conv7_b, conv8_w, conv8_b,
           conv9_w, conv9_b, conv10_w, conv10_b, conv11_w, conv11_b,
           conv12_w, conv12_b, fc_w1, fc_b1, fc_w2, fc_b2):
    ws = [conv0_w, conv1_w, conv2_w, conv3_w, conv4_w, conv5_w, conv6_w,
          conv7_w, conv8_w, conv9_w, conv10_w, conv11_w, conv12_w]
    bs = [conv0_b, conv1_b, conv2_b, conv3_b, conv4_b, conv5_b, conv6_b,
          conv7_b, conv8_b, conv9_b, conv10_b, conv11_b, conv12_b]
    x = _prep_x(x_nchw)
    for i, (hh, cout, pool, btile, pre) in enumerate(_CFG):
        if i == _TRUNC:
            return x[1, :128, :10]
        wcat = _prep_w(ws[i], pre)
        if i == len(_CFG) - 1:
            logits = _conv_fc_call(x, wcat, bs[i], fc_w1, fc_b1, fc_w2, fc_b2,
                                   btile=btile)
            return logits[:, :_NCLS]
        x = _conv_call(x, wcat, bs[i], hh=hh, cout=cout,
                       pool=pool, btile=btile, pre=pre)


# speedup vs baseline: 2.9430x; 1.4205x over previous
import functools

import jax
import jax.numpy as jnp
from jax.experimental import pallas as pl
from jax.experimental.pallas import tpu as pltpu

_NCLS = 10


def _conv_row(x_ref, w_ref, b_ref, r):
    xw = x_ref[pl.ds(r - 1, 3)]
    parts = []
    for dy in range(3):
        x0 = xw[dy]
        parts.extend([jnp.roll(x0, 1, axis=0), x0, jnp.roll(x0, -1, axis=0)])
    xc = jnp.concatenate(parts, axis=1)
    acc = jnp.dot(xc, w_ref[...], preferred_element_type=jnp.float32)
    return jnp.maximum(acc + b_ref[...], 0.0)


def _conv_first_row(x_ref, w_ref, b_ref, r, cout):
    xc = x_ref[pl.ds(r - 1, 1)][0]
    y = jnp.dot(xc, w_ref[...], preferred_element_type=jnp.float32)
    acc = (jnp.roll(y[:, :cout], 1, axis=0) + y[:, cout:2 * cout]
           + jnp.roll(y[:, 2 * cout:], -1, axis=0))
    return jnp.maximum(acc + b_ref[...], 0.0)


def _conv_plain_kernel(x_ref, w_ref, b_ref, o_ref, *, hh, wp, first):
    tm = x_ref.shape[1]
    cout = o_ref.shape[2]
    col = jax.lax.broadcasted_iota(jnp.int32, (tm, 1), 0) % wp
    valid = jnp.logical_and(col > 0, col < wp - 1)
    zrow = jnp.zeros((1, tm, cout), o_ref.dtype)
    o_ref[pl.ds(0, 1)] = zrow
    o_ref[pl.ds(hh + 1, 1)] = zrow

    for r in range(1, hh + 1):
        if first:
            acc = _conv_first_row(x_ref, w_ref, b_ref, r, cout)
        else:
            acc = _conv_row(x_ref, w_ref, b_ref, r)
        out = jnp.where(valid, acc, 0.0).astype(o_ref.dtype)
        o_ref[pl.ds(r, 1)] = out[None]


def _conv_pool_kernel(x_ref, w_ref, b_ref, s_ref, o_ref, *, hh):
    cout = o_ref.shape[2]
    tmo = o_ref.shape[1]
    ho = hh // 2
    zrow = jnp.zeros((1, tmo, cout), o_ref.dtype)
    o_ref[pl.ds(0, 1)] = zrow
    o_ref[pl.ds(ho + 1, 1)] = zrow

    ps = []
    for i in range(ho):
        c0 = _conv_row(x_ref, w_ref, b_ref, 2 * i + 1)
        c1 = _conv_row(x_ref, w_ref, b_ref, 2 * i + 2)
        v = jnp.maximum(c0, c1)
        p = jnp.maximum(v, jnp.roll(v, -1, axis=0))
        ps.append(p.astype(jnp.bfloat16))
    pcat = jnp.concatenate(ps, axis=1)
    ocat = jnp.dot(s_ref[...], pcat, preferred_element_type=jnp.float32)
    for i in range(ho):
        blk = ocat[:, i * cout:(i + 1) * cout].astype(o_ref.dtype)
        o_ref[pl.ds(i + 1, 1)] = blk[None]


def _conv_fc_kernel(x_ref, w_ref, b_ref, s_ref, w1_ref, b1_ref, w2_ref,
                    b2_ref, o_ref):
    c0 = _conv_row(x_ref, w_ref, b_ref, 1)
    c1 = _conv_row(x_ref, w_ref, b_ref, 2)
    v = jnp.maximum(c0, c1)
    p = jnp.maximum(v, jnp.roll(v, -1, axis=0)).astype(jnp.bfloat16)
    feat = jnp.dot(s_ref[...], p,
                   preferred_element_type=jnp.float32).astype(jnp.bfloat16)
    h = jnp.dot(feat, w1_ref[...], preferred_element_type=jnp.float32)
    h = jnp.maximum(h + b1_ref[...], 0.0).astype(jnp.bfloat16)
    logits = jnp.dot(h, w2_ref[...], preferred_element_type=jnp.float32)
    o_ref[...] = logits + b2_ref[...]


_CFG = [
    (32, 128, False, 16, True),
    (32, 128, True, 16, False),
    (16, 128, False, 16, False),
    (16, 128, True, 16, False),
    (8, 256, False, 32, False),
    (8, 256, False, 32, False),
    (8, 256, True, 32, False),
    (4, 512, False, 32, False),
    (4, 512, False, 32, False),
    (4, 512, True, 32, False),
    (2, 512, False, 32, False),
    (2, 512, False, 32, False),
    (2, 512, True, 32, False),
]

_VMEM = dict(vmem_limit_bytes=64 * 1024 * 1024)


def _pool_selector(nb, wp, btile):
    del nb
    wo = (wp - 2) // 2
    wpn = wo + 2
    tm = btile * wp
    tmo = btile * wpn
    j = jnp.arange(tmo)
    bo, jo = j // wpn, j % wpn
    msrc = bo * wp + 2 * jo - 1
    valid = jnp.logical_and(jo >= 1, jo <= wo)
    s = jnp.logical_and(jnp.arange(tm)[None, :] == msrc[:, None],
                        valid[:, None])
    return s.astype(jnp.bfloat16)


def _conv_call(x, wcat, bias, *, hh, cout, pool, btile, first=False):
    hp, m, _ = x.shape
    wp = hh + 2
    nb = m // wp
    btile = min(btile, nb)
    n_bt = nb // btile
    tm = btile * wp
    in_specs = [
        pl.BlockSpec((hp, tm, x.shape[2]), lambda i: (0, i, 0)),
        pl.BlockSpec(wcat.shape, lambda i: (0, 0)),
        pl.BlockSpec(bias.shape, lambda i: (0, 0)),
    ]
    args = [x, wcat, bias]
    if pool:
        ho = hh // 2
        wpn = (wp - 2) // 2 + 2
        sel = _pool_selector(nb, wp, btile)
        in_specs.append(pl.BlockSpec(sel.shape, lambda i: (0, 0)))
        args.append(sel)
        out_shape = jax.ShapeDtypeStruct((ho + 2, nb * wpn, cout), jnp.bfloat16)
        out_spec = pl.BlockSpec((ho + 2, btile * wpn, cout), lambda i: (0, i, 0))
        kern = functools.partial(_conv_pool_kernel, hh=hh)
    else:
        out_shape = jax.ShapeDtypeStruct((hh + 2, m, cout), jnp.bfloat16)
        out_spec = pl.BlockSpec((hh + 2, tm, cout), lambda i: (0, i, 0))
        kern = functools.partial(_conv_plain_kernel, hh=hh, wp=wp, first=first)
    return pl.pallas_call(
        kern,
        out_shape=out_shape,
        grid=(n_bt,),
        in_specs=in_specs,
        out_specs=out_spec,
        compiler_params=pltpu.CompilerParams(
            dimension_semantics=("parallel",), **_VMEM),
    )(*args)


def _conv_fc_call(x, wcat, bias, w1, b1, w2, b2, *, btile):
    hp, m, cin = x.shape
    wp = 4
    nb = m // wp
    btile = min(btile, nb)
    n_bt = nb // btile
    tm = btile * wp
    j = jnp.arange(btile)
    sel = (jnp.arange(tm)[None, :] == (j * wp + 1)[:, None]).astype(jnp.bfloat16)
    ncp = w2.shape[1]
    return pl.pallas_call(
        _conv_fc_kernel,
        out_shape=jax.ShapeDtypeStruct((nb, ncp), jnp.float32),
        grid=(n_bt,),
        in_specs=[
            pl.BlockSpec((hp, tm, cin), lambda i: (0, i, 0)),
            pl.BlockSpec(wcat.shape, lambda i: (0, 0)),
            pl.BlockSpec(bias.shape, lambda i: (0, 0)),
            pl.BlockSpec(sel.shape, lambda i: (0, 0)),
            pl.BlockSpec(w1.shape, lambda i: (0, 0)),
            pl.BlockSpec(b1.shape, lambda i: (0, 0)),
            pl.BlockSpec(w2.shape, lambda i: (0, 0)),
            pl.BlockSpec(b2.shape, lambda i: (0, 0)),
        ],
        out_specs=pl.BlockSpec((btile, ncp), lambda i: (i, 0)),
        compiler_params=pltpu.CompilerParams(
            dimension_semantics=("parallel",), **_VMEM),
    )(x, wcat, bias, sel, w1, b1, w2, b2)


def _prep_x(x_nchw):
    b = x_nchw.shape[0]
    x = jnp.transpose(x_nchw, (0, 2, 3, 1)).astype(jnp.bfloat16)
    x = jnp.pad(x, ((0, 0), (0, 0), (1, 1), (0, 5)))
    x = jnp.transpose(x, (1, 0, 2, 3)).reshape(32, b * 34, 8)
    x = jnp.pad(x, ((1, 1), (0, 0), (0, 0)))
    return jnp.concatenate([x[0:32], x[1:33], x[2:34]], axis=-1)


def _prep_w_first(w, cout):
    w = jnp.pad(w, ((0, 0), (0, 5), (0, 0)))
    w = w.reshape(3, 3, 8, cout).transpose(0, 2, 1, 3)
    return w.reshape(24, 3 * cout)


def kernel(x_nchw, conv0_w, conv0_b, conv1_w, conv1_b, conv2_w, conv2_b,
           conv3_w, conv3_b, conv4_w, conv4_b, conv5_w, conv5_b,
           conv6_w, conv6_b, conv7_w, conv7_b, conv8_w, conv8_b,
           conv9_w, conv9_b, conv10_w, conv10_b, conv11_w, conv11_b,
           conv12_w, conv12_b, fc_w1, fc_b1, fc_w2, fc_b2):
    ws = [conv0_w, conv1_w, conv2_w, conv3_w, conv4_w, conv5_w, conv6_w,
          conv7_w, conv8_w, conv9_w, conv10_w, conv11_w, conv12_w]
    bs = [conv0_b, conv1_b, conv2_b, conv3_b, conv4_b, conv5_b, conv6_b,
          conv7_b, conv8_b, conv9_b, conv10_b, conv11_b, conv12_b]
    x = _prep_x(x_nchw)
    for i, (hh, cout, pool, btile, first) in enumerate(_CFG):
        if first:
            wcat = _prep_w_first(ws[i], cout)
        else:
            wcat = ws[i].reshape(9 * ws[i].shape[1], ws[i].shape[2])
        if i == len(_CFG) - 1:
            logits = _conv_fc_call(x, wcat, bs[i], fc_w1, fc_b1, fc_w2, fc_b2,
                                   btile=btile)
            return logits[:, :_NCLS]
        x = _conv_call(x, wcat, bs[i], hh=hh, cout=cout,
                       pool=pool, btile=btile, first=first)


# speedup vs baseline: 2.9717x; 1.0098x over previous
import functools

import jax
import jax.numpy as jnp
from jax.experimental import pallas as pl
from jax.experimental.pallas import tpu as pltpu

_NCLS = 10


def _shifted(x_ref):
    x = x_ref[...]
    return jnp.roll(x, 1, axis=1), x, jnp.roll(x, -1, axis=1)


def _conv_row(shifted, w_ref, b_ref, r):
    rp, x0, rm = shifted
    parts = []
    for dy in range(3):
        i = r - 1 + dy
        parts.extend([rp[i], x0[i], rm[i]])
    xc = jnp.concatenate(parts, axis=1)
    acc = jnp.dot(xc, w_ref[...], preferred_element_type=jnp.float32)
    return jnp.maximum(acc + b_ref[...], 0.0)


def _conv_first_row(x_ref, w_ref, b_ref, r, cout):
    xc = x_ref[pl.ds(r - 1, 1)][0]
    y = jnp.dot(xc, w_ref[...], preferred_element_type=jnp.float32)
    acc = (jnp.roll(y[:, :cout], 1, axis=0) + y[:, cout:2 * cout]
           + jnp.roll(y[:, 2 * cout:], -1, axis=0))
    return jnp.maximum(acc + b_ref[...], 0.0)


def _conv_plain_kernel(x_ref, w_ref, b_ref, o_ref, *, hh, wp, first):
    tm = x_ref.shape[1]
    cout = o_ref.shape[2]
    col = jax.lax.broadcasted_iota(jnp.int32, (tm, 1), 0) % wp
    valid = jnp.logical_and(col > 0, col < wp - 1)
    zrow = jnp.zeros((1, tm, cout), o_ref.dtype)
    o_ref[pl.ds(0, 1)] = zrow
    o_ref[pl.ds(hh + 1, 1)] = zrow

    shifted = None if first else _shifted(x_ref)
    for r in range(1, hh + 1):
        if first:
            acc = _conv_first_row(x_ref, w_ref, b_ref, r, cout)
        else:
            acc = _conv_row(shifted, w_ref, b_ref, r)
        out = jnp.where(valid, acc, 0.0).astype(o_ref.dtype)
        o_ref[pl.ds(r, 1)] = out[None]


def _conv_pool_kernel(x_ref, w_ref, b_ref, s_ref, o_ref, *, hh):
    cout = o_ref.shape[2]
    tmo = o_ref.shape[1]
    ho = hh // 2
    zrow = jnp.zeros((1, tmo, cout), o_ref.dtype)
    o_ref[pl.ds(0, 1)] = zrow
    o_ref[pl.ds(ho + 1, 1)] = zrow

    shifted = _shifted(x_ref)
    ps = []
    for i in range(ho):
        c0 = _conv_row(shifted, w_ref, b_ref, 2 * i + 1)
        c1 = _conv_row(shifted, w_ref, b_ref, 2 * i + 2)
        v = jnp.maximum(c0, c1)
        p = jnp.maximum(v, jnp.roll(v, -1, axis=0))
        ps.append(p.astype(jnp.bfloat16))
    pcat = jnp.concatenate(ps, axis=1)
    ocat = jnp.dot(s_ref[...], pcat, preferred_element_type=jnp.float32)
    for i in range(ho):
        blk = ocat[:, i * cout:(i + 1) * cout].astype(o_ref.dtype)
        o_ref[pl.ds(i + 1, 1)] = blk[None]


def _conv_fc_kernel(x_ref, w_ref, b_ref, s_ref, w1_ref, b1_ref, w2_ref,
                    b2_ref, o_ref):
    shifted = _shifted(x_ref)
    c0 = _conv_row(shifted, w_ref, b_ref, 1)
    c1 = _conv_row(shifted, w_ref, b_ref, 2)
    v = jnp.maximum(c0, c1)
    p = jnp.maximum(v, jnp.roll(v, -1, axis=0)).astype(jnp.bfloat16)
    feat = jnp.dot(s_ref[...], p,
                   preferred_element_type=jnp.float32).astype(jnp.bfloat16)
    h = jnp.dot(feat, w1_ref[...], preferred_element_type=jnp.float32)
    h = jnp.maximum(h + b1_ref[...], 0.0).astype(jnp.bfloat16)
    logits = jnp.dot(h, w2_ref[...], preferred_element_type=jnp.float32)
    o_ref[...] = logits + b2_ref[...]


_CFG = [
    (32, 128, False, 16, True),
    (32, 128, True, 16, False),
    (16, 128, False, 16, False),
    (16, 128, True, 16, False),
    (8, 256, False, 32, False),
    (8, 256, False, 32, False),
    (8, 256, True, 32, False),
    (4, 512, False, 32, False),
    (4, 512, False, 32, False),
    (4, 512, True, 32, False),
    (2, 512, False, 32, False),
    (2, 512, False, 32, False),
    (2, 512, True, 32, False),
]

_VMEM = dict(vmem_limit_bytes=64 * 1024 * 1024)


def _pool_selector(nb, wp, btile):
    del nb
    wo = (wp - 2) // 2
    wpn = wo + 2
    tm = btile * wp
    tmo = btile * wpn
    j = jnp.arange(tmo)
    bo, jo = j // wpn, j % wpn
    msrc = bo * wp + 2 * jo - 1
    valid = jnp.logical_and(jo >= 1, jo <= wo)
    s = jnp.logical_and(jnp.arange(tm)[None, :] == msrc[:, None],
                        valid[:, None])
    return s.astype(jnp.bfloat16)


def _conv_call(x, wcat, bias, *, hh, cout, pool, btile, first=False):
    hp, m, _ = x.shape
    wp = hh + 2
    nb = m // wp
    btile = min(btile, nb)
    n_bt = nb // btile
    tm = btile * wp
    in_specs = [
        pl.BlockSpec((hp, tm, x.shape[2]), lambda i: (0, i, 0)),
        pl.BlockSpec(wcat.shape, lambda i: (0, 0)),
        pl.BlockSpec(bias.shape, lambda i: (0, 0)),
    ]
    args = [x, wcat, bias]
    if pool:
        ho = hh // 2
        wpn = (wp - 2) // 2 + 2
        sel = _pool_selector(nb, wp, btile)
        in_specs.append(pl.BlockSpec(sel.shape, lambda i: (0, 0)))
        args.append(sel)
        out_shape = jax.ShapeDtypeStruct((ho + 2, nb * wpn, cout), jnp.bfloat16)
        out_spec = pl.BlockSpec((ho + 2, btile * wpn, cout), lambda i: (0, i, 0))
        kern = functools.partial(_conv_pool_kernel, hh=hh)
    else:
        out_shape = jax.ShapeDtypeStruct((hh + 2, m, cout), jnp.bfloat16)
        out_spec = pl.BlockSpec((hh + 2, tm, cout), lambda i: (0, i, 0))
        kern = functools.partial(_conv_plain_kernel, hh=hh, wp=wp, first=first)
    return pl.pallas_call(
        kern,
        out_shape=out_shape,
        grid=(n_bt,),
        in_specs=in_specs,
        out_specs=out_spec,
        compiler_params=pltpu.CompilerParams(
            dimension_semantics=("parallel",), **_VMEM),
    )(*args)


def _conv_fc_call(x, wcat, bias, w1, b1, w2, b2, *, btile):
    hp, m, cin = x.shape
    wp = 4
    nb = m // wp
    btile = min(btile, nb)
    n_bt = nb // btile
    tm = btile * wp
    j = jnp.arange(btile)
    sel = (jnp.arange(tm)[None, :] == (j * wp + 1)[:, None]).astype(jnp.bfloat16)
    ncp = w2.shape[1]
    return pl.pallas_call(
        _conv_fc_kernel,
        out_shape=jax.ShapeDtypeStruct((nb, ncp), jnp.float32),
        grid=(n_bt,),
        in_specs=[
            pl.BlockSpec((hp, tm, cin), lambda i: (0, i, 0)),
            pl.BlockSpec(wcat.shape, lambda i: (0, 0)),
            pl.BlockSpec(bias.shape, lambda i: (0, 0)),
            pl.BlockSpec(sel.shape, lambda i: (0, 0)),
            pl.BlockSpec(w1.shape, lambda i: (0, 0)),
            pl.BlockSpec(b1.shape, lambda i: (0, 0)),
            pl.BlockSpec(w2.shape, lambda i: (0, 0)),
            pl.BlockSpec(b2.shape, lambda i: (0, 0)),
        ],
        out_specs=pl.BlockSpec((btile, ncp), lambda i: (i, 0)),
        compiler_params=pltpu.CompilerParams(
            dimension_semantics=("parallel",), **_VMEM),
    )(x, wcat, bias, sel, w1, b1, w2, b2)


def _prep_x(x_nchw):
    b = x_nchw.shape[0]
    x = jnp.transpose(x_nchw, (0, 2, 3, 1)).astype(jnp.bfloat16)
    x = jnp.pad(x, ((0, 0), (0, 0), (1, 1), (0, 5)))
    x = jnp.transpose(x, (1, 0, 2, 3)).reshape(32, b * 34, 8)
    x = jnp.pad(x, ((1, 1), (0, 0), (0, 0)))
    return jnp.concatenate([x[0:32], x[1:33], x[2:34]], axis=-1)


def _prep_w_first(w, cout):
    w = jnp.pad(w, ((0, 0), (0, 5), (0, 0)))
    w = w.reshape(3, 3, 8, cout).transpose(0, 2, 1, 3)
    return w.reshape(24, 3 * cout)


def kernel(x_nchw, conv0_w, conv0_b, conv1_w, conv1_b, conv2_w, conv2_b,
           conv3_w, conv3_b, conv4_w, conv4_b, conv5_w, conv5_b,
           conv6_w, conv6_b, conv7_w, conv7_b, conv8_w, conv8_b,
           conv9_w, conv9_b, conv10_w, conv10_b, conv11_w, conv11_b,
           conv12_w, conv12_b, fc_w1, fc_b1, fc_w2, fc_b2):
    ws = [conv0_w, conv1_w, conv2_w, conv3_w, conv4_w, conv5_w, conv6_w,
          conv7_w, conv8_w, conv9_w, conv10_w, conv11_w, conv12_w]
    bs = [conv0_b, conv1_b, conv2_b, conv3_b, conv4_b, conv5_b, conv6_b,
          conv7_b, conv8_b, conv9_b, conv10_b, conv11_b, conv12_b]
    x = _prep_x(x_nchw)
    for i, (hh, cout, pool, btile, first) in enumerate(_CFG):
        if first:
            wcat = _prep_w_first(ws[i], cout)
        else:
            wcat = ws[i].reshape(9 * ws[i].shape[1], ws[i].shape[2])
        if i == len(_CFG) - 1:
            logits = _conv_fc_call(x, wcat, bs[i], fc_w1, fc_b1, fc_w2, fc_b2,
                                   btile=btile)
            return logits[:, :_NCLS]
        x = _conv_call(x, wcat, bs[i], hh=hh, cout=cout,
                       pool=pool, btile=btile, first=first)
